# Initial kernel scaffold; baseline (speedup 1.0000x reference)
import jax
import jax.numpy as jnp
from jax.experimental import pallas as pl


def kernel(X, L_rows, L_cols, L_vals, W, b):
    raise NotImplementedError("write your pallas kernel here")


# speedup vs baseline: 4.5145x; 4.5145x over previous
import functools

import jax
import jax.numpy as jnp
from jax import lax
from jax.experimental import pallas as pl
from jax.experimental.pallas import tpu as pltpu
from jax.experimental.pallas import tpu_sc as plsc

N = 10000
E = 320000
D = 128

NC = 2
NS = 16
LANES = 16

NW = NC * NS
EDGES_PER_W = E // NW
CHUNK = 80
NCHUNKS = EDGES_PER_W // CHUNK
ROWS_PER_TILE = N // NS


def _sc_body(x_hbm, rows_hbm, cols_hbm, vals_hbm, out_hbm,
             cols_v, rows_v, vals_v, gbuf, agg, sem):
    c = lax.axis_index("c")
    s = lax.axis_index("s")
    w = c * NS + s

    zero16 = jnp.zeros((LANES,), jnp.float32)

    def _zrow(r, carry):
        for k in range(D // LANES):
            gbuf[r, pl.ds(k * LANES, LANES)] = zero16
        return carry

    lax.fori_loop(0, CHUNK, _zrow, 0)

    nrow_chunks = N // CHUNK
    for i in range((nrow_chunks + NS - 1) // NS):
        cid = s + i * NS

        @pl.when(cid < nrow_chunks)
        def _zero_chunk():
            r0 = pl.multiple_of(cid * CHUNK, CHUNK)
            pltpu.sync_copy(gbuf, agg.at[pl.ds(r0, CHUNK)])

    plsc.subcore_barrier()

    def _chunk(i, carry):
        off = pl.multiple_of(w * EDGES_PER_W + i * CHUNK, CHUNK)
        pltpu.sync_copy(cols_hbm.at[pl.ds(off, CHUNK)], cols_v)
        pltpu.sync_copy(rows_hbm.at[pl.ds(off, CHUNK)], rows_v)
        pltpu.sync_copy(vals_hbm.at[pl.ds(off, CHUNK)], vals_v)
        pltpu.async_copy(x_hbm.at[cols_v], gbuf, sem).wait()
        dnums = lax.GatherDimensionNumbers(
            offset_dims=(), collapsed_slice_dims=(0,), start_index_map=(0,))
        for g in range(CHUNK // LANES):
            vv = vals_v[pl.ds(g * LANES, LANES)]
            for j in range(LANES):
                bc = lax.gather(
                    vv, jnp.full((LANES, 1), j, jnp.int32), dnums, (1,),
                    mode=lax.GatherScatterMode.PROMISE_IN_BOUNDS)
                r = g * LANES + j
                for k in range(D // LANES):
                    sl = pl.ds(k * LANES, LANES)
                    gbuf[r, sl] = gbuf[r, sl] * bc
        pltpu.sync_copy(gbuf, agg.at[rows_v], add=True)
        return carry

    lax.fori_loop(0, NCHUNKS, _chunk, 0)
    plsc.subcore_barrier()

    for i in range((nrow_chunks + NS - 1) // NS):
        cid = s + i * NS

        @pl.when(cid < nrow_chunks)
        def _copy_chunk():
            r0 = pl.multiple_of(cid * CHUNK, CHUNK)
            pltpu.sync_copy(agg.at[pl.ds(r0, CHUNK)],
                            out_hbm.at[c, pl.ds(r0, CHUNK)])


_sc_segment_sum = functools.partial(
    pl.kernel,
    out_type=jax.ShapeDtypeStruct((NC, N, D), jnp.float32),
    mesh=plsc.VectorSubcoreMesh(core_axis_name="c", subcore_axis_name="s"),
    scratch_types=[
        pltpu.VMEM((CHUNK,), jnp.int32),
        pltpu.VMEM((CHUNK,), jnp.int32),
        pltpu.VMEM((CHUNK,), jnp.float32),
        pltpu.VMEM((CHUNK, D), jnp.float32),
        pltpu.VMEM_SHARED((N, D), jnp.float32),
        pltpu.SemaphoreType.DMA,
    ],
)(_sc_body)


BLK = 1000


def _tc_linear_body(p0_ref, p1_ref, wt_ref, b_ref, o_ref):
    acc = p0_ref[...] + p1_ref[...]
    o_ref[...] = (
        jnp.dot(acc, wt_ref[...], preferred_element_type=jnp.float32)
        + b_ref[...]
    )


def _tc_linear(p0, p1, wt, b2):
    return pl.pallas_call(
        _tc_linear_body,
        grid=(N // BLK,),
        in_specs=[
            pl.BlockSpec((BLK, D), lambda i: (i, 0)),
            pl.BlockSpec((BLK, D), lambda i: (i, 0)),
            pl.BlockSpec((D, D), lambda i: (0, 0)),
            pl.BlockSpec((1, D), lambda i: (0, 0)),
        ],
        out_specs=pl.BlockSpec((BLK, D), lambda i: (i, 0)),
        out_shape=jax.ShapeDtypeStruct((N, D), jnp.float32),
    )(p0, p1, wt, b2)


def kernel(X, L_rows, L_cols, L_vals, W, b):
    partials = _sc_segment_sum(X, L_rows, L_cols, L_vals)
    return _tc_linear(partials[0], partials[1], W.T, b.reshape(1, D))
